# baseline (speedup 1.0000x reference)
import jax
import jax.numpy as jnp
from jax.experimental import pallas as pl
from jax.experimental.pallas import tpu as pltpu

N = 1024
NB = 4
B = N // NB

_TDIMS = (((0,), (0,)), ((), ()))


def _aggT(adjv, y):
    z = jax.lax.dot_general(adjv, y, _TDIMS,
                            preferred_element_type=jnp.float32)
    return z + 2.0 * y


def _fused_kernel(adj_hbm, emb_ref, w1_ref, b1_ref, w2_ref, b2_ref,
                  fc1w_ref, fc1b_ref, fc2w_ref, fc2b_ref,
                  x_out_ref, a2_out_ref, adj_v, sem):
    for i in range(NB):
        pltpu.make_async_copy(adj_hbm.at[pl.ds(i * B, B), :],
                              adj_v.at[pl.ds(i * B, B), :],
                              sem.at[i]).start()

    xt = jnp.dot(emb_ref[...], w1_ref[...], preferred_element_type=jnp.float32)

    ones_b = jnp.ones((B, 1), jnp.float32)
    deg = jnp.full((N, 1), 2.0, jnp.float32)
    for i in range(NB):
        pltpu.make_async_copy(adj_hbm.at[pl.ds(i * B, B), :],
                              adj_v.at[pl.ds(i * B, B), :],
                              sem.at[i]).wait()
        deg = deg + jax.lax.dot_general(adj_v[pl.ds(i * B, B), :], ones_b,
                                        _TDIMS,
                                        preferred_element_type=jnp.float32)
    dis = jax.lax.rsqrt(deg)

    adjv = adj_v[...]
    x = jnp.maximum(dis * _aggT(adjv, dis * xt) + b1_ref[...], 0.0)

    xt2 = jnp.dot(x, w2_ref[...], preferred_element_type=jnp.float32)
    x2 = jnp.maximum(dis * _aggT(adjv, dis * xt2) + b2_ref[...], 0.0)
    x_out_ref[...] = x2

    h = jnp.maximum(jnp.dot(x2, fc1w_ref[...], preferred_element_type=jnp.float32)
                    + fc1b_ref[...], 0.0)
    a2_out_ref[...] = (jnp.dot(h, fc2w_ref[...], preferred_element_type=jnp.float32)
                       + fc2b_ref[...])


def kernel(adj, node_emb, W1, b1, W2, b2, fc1_W, fc1_b, fc2_W, fc2_b):
    x, a2 = pl.pallas_call(
        _fused_kernel,
        in_specs=[
            pl.BlockSpec(memory_space=pl.ANY),
            pl.BlockSpec(memory_space=pltpu.MemorySpace.VMEM),
            pl.BlockSpec(memory_space=pltpu.MemorySpace.VMEM),
            pl.BlockSpec(memory_space=pltpu.MemorySpace.VMEM),
            pl.BlockSpec(memory_space=pltpu.MemorySpace.VMEM),
            pl.BlockSpec(memory_space=pltpu.MemorySpace.VMEM),
            pl.BlockSpec(memory_space=pltpu.MemorySpace.VMEM),
            pl.BlockSpec(memory_space=pltpu.MemorySpace.VMEM),
            pl.BlockSpec(memory_space=pltpu.MemorySpace.VMEM),
            pl.BlockSpec(memory_space=pltpu.MemorySpace.VMEM),
        ],
        out_shape=(
            jax.ShapeDtypeStruct((N, 128), jnp.float32),
            jax.ShapeDtypeStruct((N, 1), jnp.float32),
        ),
        scratch_shapes=[
            pltpu.VMEM((N, N), jnp.float32),
            pltpu.SemaphoreType.DMA((NB,)),
        ],
    )(adj, node_emb, W1, b1.reshape(1, 512), W2, b2.reshape(1, 128),
      fc1_W, fc1_b.reshape(1, 256), fc2_W, fc2_b.reshape(1, 1))
    return (x, a2)
